# Initial kernel scaffold; baseline (speedup 1.0000x reference)
import jax
import jax.numpy as jnp
from jax.experimental import pallas as pl


def kernel(x_lit, x_cls, edge_index_lit_to_cls, edge_index_cls_to_lit, h_lit, c_lit, h_cls, c_cls, W_ih_lit, W_hh_lit, b_ih_lit, b_hh_lit, W_ih_cls, W_hh_cls, b_ih_cls, b_hh_cls):
    raise NotImplementedError("write your pallas kernel here")


# speedup vs baseline: 1.5514x; 1.5514x over previous
import jax
import jax.numpy as jnp
from jax.experimental import pallas as pl
from jax.experimental.pallas import tpu as pltpu

_BLOCK = 1000


def _lstm_block_kernel(x_ref, h_ref, c_ref, wih_ref, whh_ref, b_ref, out_ref):
    x = x_ref[0]
    h = h_ref[0]
    c = c_ref[0]
    gates = (
        jnp.dot(x, wih_ref[0], preferred_element_type=jnp.float32)
        + jnp.dot(h, whh_ref[0], preferred_element_type=jnp.float32)
        + b_ref[0, 0]
    )
    d = x.shape[1]
    i = jax.nn.sigmoid(gates[:, 0:d])
    f = jax.nn.sigmoid(gates[:, d : 2 * d])
    g = jnp.tanh(gates[:, 2 * d : 3 * d])
    o = jax.nn.sigmoid(gates[:, 3 * d : 4 * d])
    c_new = f * c + i * g
    h_new = o * jnp.tanh(c_new)
    out_ref[0, 0] = h_new
    out_ref[0, 1] = c_new


def kernel(x_lit, x_cls, edge_index_lit_to_cls, edge_index_cls_to_lit,
           h_lit, c_lit, h_cls, c_cls,
           W_ih_lit, W_hh_lit, b_ih_lit, b_hh_lit,
           W_ih_cls, W_hh_cls, b_ih_cls, b_hh_cls):
    del edge_index_lit_to_cls, edge_index_cls_to_lit
    n, d = x_lit.shape
    xs = jnp.stack([x_lit, x_cls])
    hs = jnp.stack([h_lit, h_cls])
    cs = jnp.stack([c_lit, c_cls])
    wih = jnp.stack([W_ih_lit.T, W_ih_cls.T])
    whh = jnp.stack([W_hh_lit.T, W_hh_cls.T])
    b = jnp.stack([b_ih_lit + b_hh_lit, b_ih_cls + b_hh_cls]).reshape(2, 1, 4 * d)

    nb = n // _BLOCK
    out = pl.pallas_call(
        _lstm_block_kernel,
        grid=(2, nb),
        in_specs=[
            pl.BlockSpec((1, _BLOCK, d), lambda i, j: (i, j, 0)),
            pl.BlockSpec((1, _BLOCK, d), lambda i, j: (i, j, 0)),
            pl.BlockSpec((1, _BLOCK, d), lambda i, j: (i, j, 0)),
            pl.BlockSpec((1, d, 4 * d), lambda i, j: (i, 0, 0)),
            pl.BlockSpec((1, d, 4 * d), lambda i, j: (i, 0, 0)),
            pl.BlockSpec((1, 1, 4 * d), lambda i, j: (i, 0, 0)),
        ],
        out_specs=pl.BlockSpec((1, 2, _BLOCK, d), lambda i, j: (i, 0, j, 0)),
        out_shape=jax.ShapeDtypeStruct((2, 2, n, d), jnp.float32),
        compiler_params=pltpu.CompilerParams(
            dimension_semantics=("arbitrary", "arbitrary"),
        ),
    )(xs, hs, cs, wih, whh, b)
    return out.reshape(4, n, d)


# speedup vs baseline: 2.5877x; 1.6680x over previous
import jax
import jax.numpy as jnp
from jax.experimental import pallas as pl
from jax.experimental.pallas import tpu as pltpu

_BLOCK = 1000


def _lstm_cell_block(x, h, c, wih, whh, b):
    gates = (
        jnp.dot(x, wih, preferred_element_type=jnp.float32)
        + jnp.dot(h, whh, preferred_element_type=jnp.float32)
        + b
    )
    d = x.shape[1]
    i = jax.nn.sigmoid(gates[:, 0:d])
    f = jax.nn.sigmoid(gates[:, d : 2 * d])
    g = jnp.tanh(gates[:, 2 * d : 3 * d])
    o = jax.nn.sigmoid(gates[:, 3 * d : 4 * d])
    c_new = f * c + i * g
    h_new = o * jnp.tanh(c_new)
    return h_new, c_new


def _both_types_kernel(xl_ref, hl_ref, cl_ref, xc_ref, hc_ref, cc_ref,
                       wihl_ref, whhl_ref, bl_ref,
                       wihc_ref, whhc_ref, bc_ref, out_ref):
    h_lit, c_lit = _lstm_cell_block(
        xl_ref[...], hl_ref[...], cl_ref[...],
        wihl_ref[...], whhl_ref[...], bl_ref[...])
    h_cls, c_cls = _lstm_cell_block(
        xc_ref[...], hc_ref[...], cc_ref[...],
        wihc_ref[...], whhc_ref[...], bc_ref[...])
    out_ref[0] = h_lit
    out_ref[1] = c_lit
    out_ref[2] = h_cls
    out_ref[3] = c_cls


def kernel(x_lit, x_cls, edge_index_lit_to_cls, edge_index_cls_to_lit,
           h_lit, c_lit, h_cls, c_cls,
           W_ih_lit, W_hh_lit, b_ih_lit, b_hh_lit,
           W_ih_cls, W_hh_cls, b_ih_cls, b_hh_cls):
    del edge_index_lit_to_cls, edge_index_cls_to_lit
    n, d = x_lit.shape
    b_lit = (b_ih_lit + b_hh_lit).reshape(1, 4 * d)
    b_cls = (b_ih_cls + b_hh_cls).reshape(1, 4 * d)

    nb = n // _BLOCK
    row_spec = pl.BlockSpec((_BLOCK, d), lambda j: (j, 0))
    w_spec = pl.BlockSpec((d, 4 * d), lambda j: (0, 0))
    b_spec = pl.BlockSpec((1, 4 * d), lambda j: (0, 0))
    out = pl.pallas_call(
        _both_types_kernel,
        grid=(nb,),
        in_specs=[
            row_spec, row_spec, row_spec,
            row_spec, row_spec, row_spec,
            w_spec, w_spec, b_spec,
            w_spec, w_spec, b_spec,
        ],
        out_specs=pl.BlockSpec((4, _BLOCK, d), lambda j: (0, j, 0)),
        out_shape=jax.ShapeDtypeStruct((4, n, d), jnp.float32),
        compiler_params=pltpu.CompilerParams(
            dimension_semantics=("arbitrary",),
        ),
    )(x_lit, h_lit, c_lit, x_cls, h_cls, c_cls,
      W_ih_lit.T, W_hh_lit.T, b_lit,
      W_ih_cls.T, W_hh_cls.T, b_cls)
    return out


# speedup vs baseline: 2.6063x; 1.0072x over previous
import jax
import jax.numpy as jnp
from jax.experimental import pallas as pl
from jax.experimental.pallas import tpu as pltpu

_BLOCK = 1000


def _lstm_cell_block(x, h, c, wih, whh, b):
    gates = (
        jnp.dot(x.astype(jnp.bfloat16), wih, preferred_element_type=jnp.float32)
        + jnp.dot(h.astype(jnp.bfloat16), whh, preferred_element_type=jnp.float32)
        + b
    )
    d = x.shape[1]
    i = jax.nn.sigmoid(gates[:, 0:d])
    f = jax.nn.sigmoid(gates[:, d : 2 * d])
    g = jnp.tanh(gates[:, 2 * d : 3 * d])
    o = jax.nn.sigmoid(gates[:, 3 * d : 4 * d])
    c_new = f * c + i * g
    h_new = o * jnp.tanh(c_new)
    return h_new, c_new


def _both_types_kernel(xl_ref, hl_ref, cl_ref, xc_ref, hc_ref, cc_ref,
                       wihl_ref, whhl_ref, bl_ref,
                       wihc_ref, whhc_ref, bc_ref, out_ref):
    h_lit, c_lit = _lstm_cell_block(
        xl_ref[...], hl_ref[...], cl_ref[...],
        wihl_ref[...], whhl_ref[...], bl_ref[...])
    h_cls, c_cls = _lstm_cell_block(
        xc_ref[...], hc_ref[...], cc_ref[...],
        wihc_ref[...], whhc_ref[...], bc_ref[...])
    out_ref[0] = h_lit
    out_ref[1] = c_lit
    out_ref[2] = h_cls
    out_ref[3] = c_cls


def kernel(x_lit, x_cls, edge_index_lit_to_cls, edge_index_cls_to_lit,
           h_lit, c_lit, h_cls, c_cls,
           W_ih_lit, W_hh_lit, b_ih_lit, b_hh_lit,
           W_ih_cls, W_hh_cls, b_ih_cls, b_hh_cls):
    del edge_index_lit_to_cls, edge_index_cls_to_lit
    n, d = x_lit.shape
    b_lit = (b_ih_lit + b_hh_lit).reshape(1, 4 * d)
    b_cls = (b_ih_cls + b_hh_cls).reshape(1, 4 * d)

    nb = n // _BLOCK
    row_spec = pl.BlockSpec((_BLOCK, d), lambda j: (j, 0))
    w_spec = pl.BlockSpec((d, 4 * d), lambda j: (0, 0))
    b_spec = pl.BlockSpec((1, 4 * d), lambda j: (0, 0))
    out = pl.pallas_call(
        _both_types_kernel,
        grid=(nb,),
        in_specs=[
            row_spec, row_spec, row_spec,
            row_spec, row_spec, row_spec,
            w_spec, w_spec, b_spec,
            w_spec, w_spec, b_spec,
        ],
        out_specs=pl.BlockSpec((4, _BLOCK, d), lambda j: (0, j, 0)),
        out_shape=jax.ShapeDtypeStruct((4, n, d), jnp.float32),
        compiler_params=pltpu.CompilerParams(
            dimension_semantics=("arbitrary",),
        ),
    )(x_lit, h_lit, c_lit, x_cls, h_cls, c_cls,
      W_ih_lit.T.astype(jnp.bfloat16), W_hh_lit.T.astype(jnp.bfloat16), b_lit,
      W_ih_cls.T.astype(jnp.bfloat16), W_hh_cls.T.astype(jnp.bfloat16), b_cls)
    return out


# speedup vs baseline: 2.6838x; 1.0297x over previous
import jax
import jax.numpy as jnp
from jax.experimental import pallas as pl
from jax.experimental.pallas import tpu as pltpu

_BLOCK = 2000


def _lstm_cell_block(x, h, c, wih, whh, b):
    gates = (
        jnp.dot(x.astype(jnp.bfloat16), wih, preferred_element_type=jnp.float32)
        + jnp.dot(h.astype(jnp.bfloat16), whh, preferred_element_type=jnp.float32)
        + b
    )
    d = x.shape[1]
    i = jax.nn.sigmoid(gates[:, 0:d])
    f = jax.nn.sigmoid(gates[:, d : 2 * d])
    g = jnp.tanh(gates[:, 2 * d : 3 * d])
    o = jax.nn.sigmoid(gates[:, 3 * d : 4 * d])
    c_new = f * c + i * g
    h_new = o * jnp.tanh(c_new)
    return h_new, c_new


def _both_types_kernel(xl_ref, hl_ref, cl_ref, xc_ref, hc_ref, cc_ref,
                       wihl_ref, whhl_ref, bl_ref,
                       wihc_ref, whhc_ref, bc_ref, out_ref):
    h_lit, c_lit = _lstm_cell_block(
        xl_ref[...], hl_ref[...], cl_ref[...],
        wihl_ref[...], whhl_ref[...], bl_ref[...])
    h_cls, c_cls = _lstm_cell_block(
        xc_ref[...], hc_ref[...], cc_ref[...],
        wihc_ref[...], whhc_ref[...], bc_ref[...])
    out_ref[0] = h_lit
    out_ref[1] = c_lit
    out_ref[2] = h_cls
    out_ref[3] = c_cls


def kernel(x_lit, x_cls, edge_index_lit_to_cls, edge_index_cls_to_lit,
           h_lit, c_lit, h_cls, c_cls,
           W_ih_lit, W_hh_lit, b_ih_lit, b_hh_lit,
           W_ih_cls, W_hh_cls, b_ih_cls, b_hh_cls):
    del edge_index_lit_to_cls, edge_index_cls_to_lit
    n, d = x_lit.shape
    b_lit = (b_ih_lit + b_hh_lit).reshape(1, 4 * d)
    b_cls = (b_ih_cls + b_hh_cls).reshape(1, 4 * d)

    nb = n // _BLOCK
    row_spec = pl.BlockSpec((_BLOCK, d), lambda j: (j, 0))
    w_spec = pl.BlockSpec((d, 4 * d), lambda j: (0, 0))
    b_spec = pl.BlockSpec((1, 4 * d), lambda j: (0, 0))
    out = pl.pallas_call(
        _both_types_kernel,
        grid=(nb,),
        in_specs=[
            row_spec, row_spec, row_spec,
            row_spec, row_spec, row_spec,
            w_spec, w_spec, b_spec,
            w_spec, w_spec, b_spec,
        ],
        out_specs=pl.BlockSpec((4, _BLOCK, d), lambda j: (0, j, 0)),
        out_shape=jax.ShapeDtypeStruct((4, n, d), jnp.float32),
        compiler_params=pltpu.CompilerParams(
            dimension_semantics=("arbitrary",),
        ),
    )(x_lit, h_lit, c_lit, x_cls, h_cls, c_cls,
      W_ih_lit.T.astype(jnp.bfloat16), W_hh_lit.T.astype(jnp.bfloat16), b_lit,
      W_ih_cls.T.astype(jnp.bfloat16), W_hh_cls.T.astype(jnp.bfloat16), b_cls)
    return out
